# baseline (speedup 1.0000x reference)
import functools

import jax
import jax.numpy as jnp
from jax import lax
from jax.experimental import pallas as pl
from jax.experimental.pallas import tpu as pltpu
from jax.experimental.pallas import tpu_sc as plsc

BLOCK = 64
NSEL = 16
LANES = 16


def kernel(keys, compression_scores):
  batch, seq_len, key_dim = keys.shape
  num_blocks = seq_len // BLOCK
  nchunks = num_blocks // LANES
  out_rows = NSEL * BLOCK
  nring = 8
  lag = 3

  info = plsc.get_sparse_core_info()
  nc, ns = 1, info.num_subcores
  batches_per_worker = batch // (nc * ns)

  table = keys.reshape(batch * seq_len, key_dim)

  mesh = plsc.VectorSubcoreMesh(core_axis_name="c", subcore_axis_name="s",
                                num_cores=nc)

  @functools.partial(
      pl.kernel,
      out_type=jax.ShapeDtypeStruct((batch * out_rows, key_dim), jnp.float32),
      mesh=mesh,
      scratch_types=[
          pltpu.VMEM((num_blocks,), jnp.float32),
          pltpu.VMEM((nring, BLOCK, key_dim), jnp.float32),
          pltpu.SemaphoreType.DMA,
          pltpu.SemaphoreType.DMA,
      ],
  )
  def run(table_hbm, scores_hbm, out_hbm, scores_v, buf, gsem, osem):
    w = lax.axis_index("s") * nc + lax.axis_index("c")

    neg_inf = jnp.float32(-jnp.inf)
    big = jnp.int32(num_blocks)
    lane = lax.iota(jnp.int32, LANES)
    perms = [lane ^ s for s in (8, 4, 2, 1)]
    gidx = [lax.iota(jnp.int32, LANES) + LANES * i for i in range(nchunks)]

    def butterfly(v, op):
      for s in range(4):
        v = op(v, v.at[perms[s]].get(mode="promise_in_bounds"))
      return v

    total = batches_per_worker * NSEL
    gathers = [None] * total
    outs = [None] * total
    bases = []

    def start_out(g):
      gathers[g].wait()
      out_base = bases[g // NSEL][1]
      j = g % NSEL
      outs[g] = pltpu.async_copy(
          buf.at[g % nring],
          out_hbm.at[pl.ds(out_base + j * BLOCK, BLOCK)], osem)

    for bb in range(batches_per_worker):
      b = w * batches_per_worker + bb
      bases.append((b * (num_blocks * BLOCK), b * out_rows))
      pltpu.sync_copy(scores_hbm.at[b], scores_v)
      chunks = [scores_v[pl.ds(LANES * i, LANES)] for i in range(nchunks)]
      valid = [jnp.ones((LANES,), jnp.bool_) for _ in range(nchunks)]
      for j in range(NSEL):
        g = bb * NSEL + j
        masked = [jnp.where(valid[i], chunks[i], neg_inf)
                  for i in range(nchunks)]
        mv = masked[0]
        for i in range(1, nchunks):
          mv = jnp.maximum(mv, masked[i])
        m = butterfly(mv, jnp.maximum)
        iv = jnp.where(valid[0] & (chunks[0] == m), gidx[0], big)
        for i in range(1, nchunks):
          iv = jnp.minimum(iv, jnp.where(valid[i] & (chunks[i] == m), gidx[i],
                                         big))
        sel_v = butterfly(iv, jnp.minimum)
        valid = [valid[i] & (gidx[i] != sel_v) for i in range(nchunks)]
        sel = sel_v[0]
        if g >= nring:
          outs[g - nring].wait()
        gathers[g] = pltpu.async_copy(
            table_hbm.at[pl.ds(bases[bb][0] + sel * BLOCK, BLOCK)],
            buf.at[g % nring], gsem)
        if g >= lag:
          start_out(g - lag)
    for g in range(total - lag, total):
      start_out(g)
    for g in range(total - nring, total):
      outs[g].wait()

  out = run(table, compression_scores)
  return out.reshape(batch, out_rows, key_dim)
